# Initial kernel scaffold; baseline (speedup 1.0000x reference)
import jax
import jax.numpy as jnp
from jax.experimental import pallas as pl


def kernel(pts_image_features, point_features, coords, W1, b1, W2, b2, W3, b3, Wp, bp, Wi, bi):
    raise NotImplementedError("write your pallas kernel here")


# speedup vs baseline: 27.8034x; 27.8034x over previous
import functools

import jax
import jax.numpy as jnp
from jax import lax
from jax.experimental import pallas as pl
from jax.experimental.pallas import tpu as pltpu
from jax.experimental.pallas import tpu_sc as plsc

N = 8192
K = 16
ROWS = 256
GRID_A = N // ROWS
NLEV = 6
NT = N // 128
FROWS = 1024
DPAD = 128

SC_NC = 2
SC_NS = 16
SC_NW = SC_NC * SC_NS
PW = N // SC_NW
CHUNK_PTS = 8
CHUNK_ROWS = CHUNK_PTS * K
CHUNKS = PW // CHUNK_PTS
IDX_ROWS_W = CHUNKS


def _knn_body(dd_ref, idx_ref):
    maxi = jnp.int32(0x7FFFFFFF)
    d = dd_ref[...]
    b = lax.bitcast_convert_type(d, jnp.int32)
    key = jnp.where(b < 0, b ^ maxi, b)

    levs = [jnp.full((ROWS, 128), maxi, jnp.int32) for _ in range(NLEV)]
    lids = [jnp.zeros((ROWS, 128), jnp.int32) for _ in range(NLEV)]
    for t in range(NT):
        p = key[:, t * 128:(t + 1) * 128]
        pid = jnp.full((ROWS, 128), t, jnp.int32)
        for v in range(NLEV):
            m = levs[v] <= p
            lo = jnp.where(m, levs[v], p)
            hi = jnp.where(m, p, levs[v])
            loid = jnp.where(m, lids[v], pid)
            hiid = jnp.where(m, pid, lids[v])
            levs[v], lids[v], p, pid = lo, loid, hi, hiid

    lane = lax.broadcasted_iota(jnp.int32, (ROWS, 128), 1)
    cols = []
    for _ in range(K):
        g = jnp.min(levs[0], axis=1, keepdims=True)
        l = jnp.min(jnp.where(levs[0] == g, lane, jnp.int32(128)),
                    axis=1, keepdims=True)
        hit = lane == l
        t_of = jnp.min(jnp.where(hit, lids[0], maxi), axis=1, keepdims=True)
        cols.append(t_of * 128 + l)
        for v in range(NLEV - 1):
            levs[v] = jnp.where(hit, levs[v + 1], levs[v])
            lids[v] = jnp.where(hit, lids[v + 1], lids[v])
        levs[NLEV - 1] = jnp.where(hit, maxi, levs[NLEV - 1])
    idx_ref[...] = jnp.concatenate(cols, axis=1)


def _final_body(p_ref, s_ref, wp_ref, ws_ref, b_ref, pt_ref, img_ref):
    hp = lax.Precision.HIGHEST
    ptb = lax.dot_general(p_ref[...], wp_ref[...], (((1,), (0,)), ((), ())),
                          precision=hp, preferred_element_type=jnp.float32)
    imgb = lax.dot_general(s_ref[...], ws_ref[...], (((1,), (0,)), ((), ())),
                           precision=hp, preferred_element_type=jnp.float32)
    pt_ref[...] = ptb + b_ref[0:1, :]
    img_ref[...] = imgb + b_ref[1:2, :]


def _sc_gather_sum_body(table_hbm, idx_hbm, out_hbm,
                        idx_v, buf0, buf1, out_v, sem0, sem1):
    wid = lax.axis_index("s") * SC_NC + lax.axis_index("c")
    base_row = wid * IDX_ROWS_W

    pltpu.sync_copy(idx_hbm.at[pl.ds(base_row, IDX_ROWS_W)], idx_v)

    coord_mask = lax.iota(jnp.int32, 16) < 3
    kf = jnp.float32(K)

    def reduce_chunk(buf, chunk):
        def point_body(p, _):
            r0 = p * K
            orow = chunk * CHUNK_PTS + p
            for v in range(DPAD // 16):
                sl = pl.ds(v * 16, 16)
                acc = buf[r0, sl]
                for r in range(1, K):
                    acc = acc + buf[r0 + r, sl]
                if v == 4:
                    acc = jnp.where(coord_mask, acc - kf * buf[r0, sl], acc)
                out_v[orow, sl] = acc
            return 0
        lax.fori_loop(0, CHUNK_PTS, point_body, 0)

    pltpu.async_copy(table_hbm.at[idx_v.at[0]], buf0, sem0)

    def pair_body(i, _):
        c0 = 2 * i
        pltpu.async_copy(table_hbm.at[idx_v.at[c0 + 1]], buf1, sem1)
        pltpu.make_async_copy(table_hbm.at[idx_v.at[c0]], buf0, sem0).wait()
        reduce_chunk(buf0, c0)

        @pl.when(i < CHUNKS // 2 - 1)
        def _():
            pltpu.async_copy(table_hbm.at[idx_v.at[c0 + 2]], buf0, sem0)

        pltpu.make_async_copy(table_hbm.at[idx_v.at[c0 + 1]], buf1, sem1).wait()
        reduce_chunk(buf1, c0 + 1)
        return 0

    lax.fori_loop(0, CHUNKS // 2, pair_body, 0)

    pltpu.sync_copy(out_v, out_hbm.at[pl.ds(wid * PW, PW)])


def _sc_gather_sum(table, idx2d):
    mesh = plsc.VectorSubcoreMesh(core_axis_name="c", subcore_axis_name="s")
    k = functools.partial(
        pl.kernel,
        mesh=mesh,
        out_type=jax.ShapeDtypeStruct((N, DPAD), jnp.float32),
        scratch_types=[
            pltpu.VMEM((IDX_ROWS_W, CHUNK_ROWS), jnp.int32),
            pltpu.VMEM((CHUNK_ROWS, DPAD), jnp.float32),
            pltpu.VMEM((CHUNK_ROWS, DPAD), jnp.float32),
            pltpu.VMEM((PW, DPAD), jnp.float32),
            pltpu.SemaphoreType.DMA,
            pltpu.SemaphoreType.DMA,
        ],
    )(_sc_gather_sum_body)
    return k(table, idx2d)


def _knn_indices(dd):
    return pl.pallas_call(
        _knn_body,
        grid=(GRID_A,),
        in_specs=[pl.BlockSpec((ROWS, N), lambda i: (i, 0))],
        out_specs=pl.BlockSpec((ROWS, K), lambda i: (i, 0)),
        out_shape=jax.ShapeDtypeStruct((N, K), jnp.int32),
    )(dd)


def _final_matmuls(pts, s, WpT, WsT, biases):
    return pl.pallas_call(
        _final_body,
        grid=(N // FROWS,),
        in_specs=[
            pl.BlockSpec((FROWS, 128), lambda i: (i, 0)),
            pl.BlockSpec((FROWS, DPAD), lambda i: (i, 0)),
            pl.BlockSpec((128, 128), lambda i: (0, 0)),
            pl.BlockSpec((DPAD, 128), lambda i: (0, 0)),
            pl.BlockSpec((8, 128), lambda i: (0, 0)),
        ],
        out_specs=[
            pl.BlockSpec((FROWS, 128), lambda i: (i, 0)),
            pl.BlockSpec((FROWS, 128), lambda i: (i, 0)),
        ],
        out_shape=[
            jax.ShapeDtypeStruct((N, 128), jnp.float32),
            jax.ShapeDtypeStruct((N, 128), jnp.float32),
        ],
    )(pts, s, WpT, WsT, biases)


def kernel(pts_image_features, point_features, coords, W1, b1, W2, b2, W3, b3,
           Wp, bp, Wi, bi):
    img = pts_image_features[0]
    pts = point_features[0]
    c = coords[0]

    x2 = jnp.sum(coords * coords, axis=-1)
    dist = x2[:, :, None] + x2[:, None, :] - 2.0 * jnp.einsum(
        'bnd,bmd->bnm', coords, coords)
    dist = jax.lax.stop_gradient(dist)

    idx = _knn_indices(dist[0])
    idx2d = idx.reshape(N * K // CHUNK_ROWS, CHUNK_ROWS)

    table = jnp.pad(jnp.concatenate([img, c], axis=1), ((0, 0), (0, 61)))
    s = _sc_gather_sum(table, idx2d)

    M = W3 @ (W2 @ W1)
    cvec = b3 + W3 @ b2 + W3 @ (W2 @ b1)
    Mf = Wi @ M
    WsT = jnp.pad(Mf, ((0, 0), (0, 61))).T
    const_img = float(K) * (Wi @ cvec) + bi
    biases = jnp.pad(jnp.stack([bp, const_img]), ((0, 6), (0, 0)))

    pt_out, img_out = _final_matmuls(pts, s, Wp.T, WsT, biases)
    return jnp.concatenate([pt_out, img_out], axis=-1)[None]


# speedup vs baseline: 34.2715x; 1.2326x over previous
import functools

import jax
import jax.numpy as jnp
from jax import lax
from jax.experimental import pallas as pl
from jax.experimental.pallas import tpu as pltpu
from jax.experimental.pallas import tpu_sc as plsc

N = 8192
K = 16
ROWS = 256
GRID_A = N // ROWS
NQ = 5
NT = N // 128
FROWS = 1024
DPAD = 128

SC_NC = 2
SC_NS = 16
SC_NW = SC_NC * SC_NS
PW = N // SC_NW
CHUNK_PTS = 8
CHUNK_ROWS = CHUNK_PTS * K
CHUNKS = PW // CHUNK_PTS
IDX_ROWS_W = CHUNKS


def _knn_body(dd_ref, idx_ref):
    maxi = jnp.int32(0x7FFFFFFF)
    d = dd_ref[...]
    b = lax.bitcast_convert_type(d, jnp.int32)
    key = jnp.where(b < 0, b ^ maxi, b)

    lev0 = jnp.full((ROWS, 128), maxi, jnp.int32)
    id0 = jnp.zeros((ROWS, 128), jnp.int32)
    plev = [jnp.full((ROWS, 128), maxi, jnp.int32) for _ in range(NQ)]
    for t in range(NT):
        p = key[:, t * 128:(t + 1) * 128]
        m = lev0 <= p
        disp = jnp.where(m, p, lev0)
        dispid = jnp.where(m, jnp.int32(t), id0)
        lev0 = jnp.minimum(lev0, p)
        id0 = jnp.where(m, id0, jnp.int32(t))
        q = jax.lax.bitwise_or(jax.lax.bitwise_and(disp, jnp.int32(-64)),
                               dispid)
        for v in range(NQ):
            lo = jnp.minimum(plev[v], q)
            q = jnp.maximum(plev[v], q)
            plev[v] = lo

    lane = lax.broadcasted_iota(jnp.int32, (ROWS, 128), 1)
    g = jnp.min(lev0, axis=1, keepdims=True)
    l = jnp.min(jnp.where(lev0 == g, lane, jnp.int32(128)),
                axis=1, keepdims=True)
    hit = lane == l
    t_of = jnp.min(jnp.where(hit, id0, maxi), axis=1, keepdims=True)
    cols = [t_of * 128 + l]
    q0 = jax.lax.bitwise_or(jax.lax.bitwise_and(lev0, jnp.int32(-64)), id0)
    state = [q0] + plev
    for v in range(NQ):
        state[v] = jnp.where(hit, state[v + 1], state[v])
    state[NQ] = jnp.where(hit, maxi, state[NQ])
    for _ in range(K - 1):
        g = jnp.min(state[0], axis=1, keepdims=True)
        l = jnp.min(jnp.where(state[0] == g, lane, jnp.int32(128)),
                    axis=1, keepdims=True)
        hit = lane == l
        cols.append(jax.lax.bitwise_and(g, jnp.int32(63)) * 128 + l)
        for v in range(NQ):
            state[v] = jnp.where(hit, state[v + 1], state[v])
        state[NQ] = jnp.where(hit, maxi, state[NQ])
    idx_ref[...] = jnp.concatenate(cols, axis=1)


def _final_body(p_ref, s_ref, wp_ref, ws_ref, b_ref, pt_ref, img_ref):
    hp = lax.Precision.HIGHEST
    ptb = lax.dot_general(p_ref[...], wp_ref[...], (((1,), (0,)), ((), ())),
                          precision=hp, preferred_element_type=jnp.float32)
    imgb = lax.dot_general(s_ref[...], ws_ref[...], (((1,), (0,)), ((), ())),
                           precision=hp, preferred_element_type=jnp.float32)
    pt_ref[...] = ptb + b_ref[0:1, :]
    img_ref[...] = imgb + b_ref[1:2, :]


def _sc_gather_sum_body(table_hbm, idx_hbm, out_hbm,
                        idx_v, buf0, buf1, out_v, sem0, sem1):
    wid = lax.axis_index("s") * SC_NC + lax.axis_index("c")
    base_row = wid * IDX_ROWS_W

    pltpu.sync_copy(idx_hbm.at[pl.ds(base_row, IDX_ROWS_W)], idx_v)

    coord_mask = lax.iota(jnp.int32, 16) < 3
    kf = jnp.float32(K)

    def reduce_chunk(buf, chunk):
        def point_body(p, _):
            r0 = p * K
            orow = chunk * CHUNK_PTS + p
            for v in range(DPAD // 16):
                sl = pl.ds(v * 16, 16)
                acc = buf[r0, sl]
                for r in range(1, K):
                    acc = acc + buf[r0 + r, sl]
                if v == 4:
                    acc = jnp.where(coord_mask, acc - kf * buf[r0, sl], acc)
                out_v[orow, sl] = acc
            return 0
        lax.fori_loop(0, CHUNK_PTS, point_body, 0)

    pltpu.async_copy(table_hbm.at[idx_v.at[0]], buf0, sem0)

    def pair_body(i, _):
        c0 = 2 * i
        pltpu.async_copy(table_hbm.at[idx_v.at[c0 + 1]], buf1, sem1)
        pltpu.make_async_copy(table_hbm.at[idx_v.at[c0]], buf0, sem0).wait()
        reduce_chunk(buf0, c0)

        @pl.when(i < CHUNKS // 2 - 1)
        def _():
            pltpu.async_copy(table_hbm.at[idx_v.at[c0 + 2]], buf0, sem0)

        pltpu.make_async_copy(table_hbm.at[idx_v.at[c0 + 1]], buf1, sem1).wait()
        reduce_chunk(buf1, c0 + 1)
        return 0

    lax.fori_loop(0, CHUNKS // 2, pair_body, 0)

    pltpu.sync_copy(out_v, out_hbm.at[pl.ds(wid * PW, PW)])


def _sc_gather_sum(table, idx2d):
    mesh = plsc.VectorSubcoreMesh(core_axis_name="c", subcore_axis_name="s")
    k = functools.partial(
        pl.kernel,
        mesh=mesh,
        out_type=jax.ShapeDtypeStruct((N, DPAD), jnp.float32),
        scratch_types=[
            pltpu.VMEM((IDX_ROWS_W, CHUNK_ROWS), jnp.int32),
            pltpu.VMEM((CHUNK_ROWS, DPAD), jnp.float32),
            pltpu.VMEM((CHUNK_ROWS, DPAD), jnp.float32),
            pltpu.VMEM((PW, DPAD), jnp.float32),
            pltpu.SemaphoreType.DMA,
            pltpu.SemaphoreType.DMA,
        ],
    )(_sc_gather_sum_body)
    return k(table, idx2d)


def _knn_indices(dd):
    return pl.pallas_call(
        _knn_body,
        grid=(GRID_A,),
        in_specs=[pl.BlockSpec((ROWS, N), lambda i: (i, 0))],
        out_specs=pl.BlockSpec((ROWS, K), lambda i: (i, 0)),
        out_shape=jax.ShapeDtypeStruct((N, K), jnp.int32),
    )(dd)


def _final_matmuls(pts, s, WpT, WsT, biases):
    return pl.pallas_call(
        _final_body,
        grid=(N // FROWS,),
        in_specs=[
            pl.BlockSpec((FROWS, 128), lambda i: (i, 0)),
            pl.BlockSpec((FROWS, DPAD), lambda i: (i, 0)),
            pl.BlockSpec((128, 128), lambda i: (0, 0)),
            pl.BlockSpec((DPAD, 128), lambda i: (0, 0)),
            pl.BlockSpec((8, 128), lambda i: (0, 0)),
        ],
        out_specs=[
            pl.BlockSpec((FROWS, 128), lambda i: (i, 0)),
            pl.BlockSpec((FROWS, 128), lambda i: (i, 0)),
        ],
        out_shape=[
            jax.ShapeDtypeStruct((N, 128), jnp.float32),
            jax.ShapeDtypeStruct((N, 128), jnp.float32),
        ],
    )(pts, s, WpT, WsT, biases)


def kernel(pts_image_features, point_features, coords, W1, b1, W2, b2, W3, b3,
           Wp, bp, Wi, bi):
    img = pts_image_features[0]
    pts = point_features[0]
    c = coords[0]

    x2 = jnp.sum(coords * coords, axis=-1)
    dist = x2[:, :, None] + x2[:, None, :] - 2.0 * jnp.einsum(
        'bnd,bmd->bnm', coords, coords)
    dist = jax.lax.stop_gradient(dist)

    idx = _knn_indices(dist[0])
    idx2d = idx.reshape(N * K // CHUNK_ROWS, CHUNK_ROWS)

    table = jnp.pad(jnp.concatenate([img, c], axis=1), ((0, 0), (0, 61)))
    s = _sc_gather_sum(table, idx2d)

    M = W3 @ (W2 @ W1)
    cvec = b3 + W3 @ b2 + W3 @ (W2 @ b1)
    Mf = Wi @ M
    WsT = jnp.pad(Mf, ((0, 0), (0, 61))).T
    const_img = float(K) * (Wi @ cvec) + bi
    biases = jnp.pad(jnp.stack([bp, const_img]), ((0, 6), (0, 0)))

    pt_out, img_out = _final_matmuls(pts, s, Wp.T, WsT, biases)
    return jnp.concatenate([pt_out, img_out], axis=-1)[None]
